# baseline (speedup 1.0000x reference)
import jax
import jax.numpy as jnp
from jax import lax
from jax.experimental import pallas as pl
from jax.experimental.pallas import tpu as pltpu
from jax.experimental.pallas import tpu_sc as plsc

D = 512
LANES = 16
KD = D // LANES
EPS = 1e-5


def _allreduce_sum(v):
    idx = lax.iota(jnp.int32, LANES)
    for sh in (8, 4, 2, 1):
        v = v + v[(idx + sh) & (LANES - 1)]
    return v


def _rsqrt_newton(v):
    bits = lax.bitcast_convert_type(v, jnp.int32)
    y = lax.bitcast_convert_type(jnp.int32(0x5F3759DF) - (bits >> 1), jnp.float32)
    for _ in range(4):
        y = y * (1.5 - 0.5 * v * y * y)
    return y


def _make_sc_kernel(n_tok, chunk, n_chunks_per_w):
    mesh = plsc.VectorSubcoreMesh(core_axis_name="c", subcore_axis_name="s")
    NW = 32
    tok_per_w = n_tok // NW

    def body(x_hbm, table_hbm, pos_hbm, mod_hbm, g_hbm, b_hbm, out_hbm,
             idx_v, rows_v, pos_v, mod_v, g_v, b_v, gsem):
        wid = lax.axis_index("s") * 2 + lax.axis_index("c")
        base = wid * tok_per_w
        pos_base = base % 8192

        pltpu.sync_copy(x_hbm.at[wid], idx_v)
        pltpu.sync_copy(mod_hbm.at[0], mod_v)
        pltpu.sync_copy(g_hbm, g_v)
        pltpu.sync_copy(b_hbm, b_v)

        def chunk_body(c, _):
            tok0 = base + c * chunk
            pltpu.async_copy(table_hbm.at[idx_v.at[c]], rows_v, gsem).wait()
            pltpu.sync_copy(pos_hbm.at[pl.ds(pos_base + c * chunk, chunk)], pos_v)

            def tok_body(j, _):
                def acc_body(k, carry):
                    acc, acc2 = carry
                    sl = pl.ds(k * LANES, LANES)
                    h = rows_v[j, sl] + pos_v[j, sl] + mod_v[sl]
                    rows_v[j, sl] = h
                    return acc + h, acc2 + h * h

                z = jnp.zeros((LANES,), jnp.float32)
                acc, acc2 = lax.fori_loop(0, KD, acc_body, (z, z))
                meanv = _allreduce_sum(acc) * (1.0 / D)
                varv = _allreduce_sum(acc2) * (1.0 / D) - meanv * meanv
                inv = _rsqrt_newton(varv + EPS)

                def norm_body(k, _):
                    sl = pl.ds(k * LANES, LANES)
                    h = rows_v[j, sl]
                    rows_v[j, sl] = (h - meanv) * inv * g_v[sl] + b_v[sl]
                    return 0

                lax.fori_loop(0, KD, norm_body, 0)
                return 0

            lax.fori_loop(0, chunk, tok_body, 0)
            pltpu.sync_copy(rows_v, out_hbm.at[pl.ds(tok0, chunk)])
            return 0

        lax.fori_loop(0, n_chunks_per_w, chunk_body, 0)

    return pl.kernel(
        body,
        out_type=jax.ShapeDtypeStruct((n_tok, D), jnp.float32),
        mesh=mesh,
        scratch_types=[
            pltpu.VMEM((n_chunks_per_w, chunk), jnp.int32),
            pltpu.VMEM((chunk, D), jnp.float32),
            pltpu.VMEM((chunk, D), jnp.float32),
            pltpu.VMEM((D,), jnp.float32),
            pltpu.VMEM((D,), jnp.float32),
            pltpu.VMEM((D,), jnp.float32),
            pltpu.SemaphoreType.DMA,
        ],
    )


def kernel(x, token_table, pos_emb, mod_table, gamma, beta):
    bsz, seq = x.shape
    n_tok = bsz * seq
    chunk = 64
    n_chunks_per_w = n_tok // 32 // chunk
    x_arr = x.astype(jnp.int32).reshape(32, n_chunks_per_w, chunk)
    pos2d = pos_emb.reshape(pos_emb.shape[1], D)
    fn = _make_sc_kernel(n_tok, chunk, n_chunks_per_w)
    out = fn(x_arr, token_table, pos2d, mod_table, gamma, beta)
    return out.reshape(bsz, seq, D)


# speedup vs baseline: 3.4319x; 3.4319x over previous
import functools

import jax
import jax.numpy as jnp
from jax import lax
from jax.experimental import pallas as pl
from jax.experimental.pallas import tpu as pltpu
from jax.experimental.pallas import tpu_sc as plsc

D = 512
EPS = 1e-5
NW = 32
CHUNK = 64


def _make_sc_gather(n_tok):
    tok_per_w = n_tok // NW
    n_chunks = tok_per_w // CHUNK
    mesh = plsc.VectorSubcoreMesh(core_axis_name="c", subcore_axis_name="s")

    def body(x_hbm, table_hbm, out_hbm, idx_v, buf0, buf1, g0, g1, o0, o1):
        wid = lax.axis_index("s") * 2 + lax.axis_index("c")
        base = wid * tok_per_w
        pltpu.sync_copy(x_hbm.at[wid], idx_v)

        bufs = (buf0, buf1)
        gsems = (g0, g1)
        osems = (o0, o1)

        def gather(c):
            return pltpu.async_copy(
                table_hbm.at[idx_v.at[c]], bufs[c % 2], gsems[c % 2])

        def put(c):
            return pltpu.async_copy(
                bufs[c % 2], out_hbm.at[pl.ds(base + c * CHUNK, CHUNK)],
                osems[c % 2])

        gathers = [None] * n_chunks
        puts = [None] * n_chunks
        gathers[0] = gather(0)
        for c in range(n_chunks):
            if c + 1 < n_chunks:
                if c >= 1:
                    puts[c - 1].wait()
                gathers[c + 1] = gather(c + 1)
            gathers[c].wait()
            puts[c] = put(c)
        puts[n_chunks - 2].wait()
        puts[n_chunks - 1].wait()

    return pl.kernel(
        body,
        out_type=jax.ShapeDtypeStruct((n_tok, D), jnp.float32),
        mesh=mesh,
        scratch_types=[
            pltpu.VMEM((n_chunks, CHUNK), jnp.int32),
            pltpu.VMEM((CHUNK, D), jnp.float32),
            pltpu.VMEM((CHUNK, D), jnp.float32),
            pltpu.SemaphoreType.DMA,
            pltpu.SemaphoreType.DMA,
            pltpu.SemaphoreType.DMA,
            pltpu.SemaphoreType.DMA,
        ],
    )


def _tc_ln_body(rows_ref, pos_ref, mod_ref, g_ref, b_ref, o_ref):
    h = rows_ref[...] + pos_ref[...] + mod_ref[...]
    mean = jnp.mean(h, axis=-1, keepdims=True)
    c = h - mean
    var = jnp.mean(c * c, axis=-1, keepdims=True)
    o_ref[...] = c * lax.rsqrt(var + EPS) * g_ref[...] + b_ref[...]


def _tc_ln(rows, pos2d, mod_row, g2d, b2d, tb, seq):
    n_tok = rows.shape[0]
    pos_blocks = seq // tb
    return pl.pallas_call(
        _tc_ln_body,
        grid=(n_tok // tb,),
        in_specs=[
            pl.BlockSpec((tb, D), lambda i: (i, 0)),
            pl.BlockSpec((tb, D), lambda i: (i % pos_blocks, 0)),
            pl.BlockSpec((1, D), lambda i: (0, 0)),
            pl.BlockSpec((1, D), lambda i: (0, 0)),
            pl.BlockSpec((1, D), lambda i: (0, 0)),
        ],
        out_specs=pl.BlockSpec((tb, D), lambda i: (i, 0)),
        out_shape=jax.ShapeDtypeStruct((n_tok, D), jnp.float32),
    )(rows, pos2d, mod_row, g2d, b2d)


def kernel(x, token_table, pos_emb, mod_table, gamma, beta):
    bsz, seq = x.shape
    n_tok = bsz * seq
    n_chunks = n_tok // NW // CHUNK
    x_arr = x.astype(jnp.int32).reshape(NW, n_chunks, CHUNK)
    rows = _make_sc_gather(n_tok)(x_arr, token_table)
    pos2d = pos_emb.reshape(seq, D)
    out = _tc_ln(rows, pos2d, mod_table[0:1], gamma.reshape(1, D),
                 beta.reshape(1, D), 256, seq)
    return out.reshape(bsz, seq, D)


# speedup vs baseline: 4.7715x; 1.3904x over previous
import functools

import jax
import jax.numpy as jnp
from jax import lax
from jax.experimental import pallas as pl
from jax.experimental.pallas import tpu as pltpu
from jax.experimental.pallas import tpu_sc as plsc

D = 512
EPS = 1e-5
NW = 32
CHUNK = 64


def _make_sc_gather(n_tok):
    tok_per_w = n_tok // NW
    n_chunks = tok_per_w // CHUNK
    mesh = plsc.VectorSubcoreMesh(core_axis_name="c", subcore_axis_name="s")

    def body(x_hbm, table_hbm, out_hbm, idx_v, buf0, buf1, g0, g1, o0, o1):
        wid = lax.axis_index("s") * 2 + lax.axis_index("c")
        base = wid * tok_per_w
        pltpu.sync_copy(x_hbm.at[wid], idx_v)

        bufs = (buf0, buf1)
        gsems = (g0, g1)
        osems = (o0, o1)

        def gather(c):
            return pltpu.async_copy(
                table_hbm.at[idx_v.at[c]], bufs[c % 2], gsems[c % 2])

        def put(c):
            return pltpu.async_copy(
                bufs[c % 2], out_hbm.at[pl.ds(base + c * CHUNK, CHUNK)],
                osems[c % 2])

        gathers = [None] * n_chunks
        puts = [None] * n_chunks
        gathers[0] = gather(0)
        for c in range(n_chunks):
            if c + 1 < n_chunks:
                if c >= 1:
                    puts[c - 1].wait()
                gathers[c + 1] = gather(c + 1)
            gathers[c].wait()
            puts[c] = put(c)
        puts[n_chunks - 2].wait()
        puts[n_chunks - 1].wait()

    return pl.kernel(
        body,
        out_type=jax.ShapeDtypeStruct((n_tok, D), jnp.float32),
        mesh=mesh,
        scratch_types=[
            pltpu.VMEM((n_chunks, CHUNK), jnp.int32),
            pltpu.VMEM((CHUNK, D), jnp.float32),
            pltpu.VMEM((CHUNK, D), jnp.float32),
            pltpu.SemaphoreType.DMA,
            pltpu.SemaphoreType.DMA,
            pltpu.SemaphoreType.DMA,
            pltpu.SemaphoreType.DMA,
        ],
    )


def _tc_ln_body(rows_ref, pos_ref, mod_ref, g_ref, b_ref, o_ref):
    h = rows_ref[...] + pos_ref[...] + mod_ref[...]
    mean = jnp.mean(h, axis=-1, keepdims=True)
    c = h - mean
    var = jnp.mean(c * c, axis=-1, keepdims=True)
    o_ref[...] = c * lax.rsqrt(var + EPS) * g_ref[...] + b_ref[...]


def _tc_ln(rows, pos2d, mod_row, g2d, b2d, tb, seq):
    n_tok = rows.shape[0]
    pos_blocks = seq // tb
    return pl.pallas_call(
        _tc_ln_body,
        grid=(n_tok // tb,),
        in_specs=[
            pl.BlockSpec((tb, D), lambda i: (i, 0)),
            pl.BlockSpec((tb, D), lambda i: (i % pos_blocks, 0)),
            pl.BlockSpec((1, D), lambda i: (0, 0)),
            pl.BlockSpec((1, D), lambda i: (0, 0)),
            pl.BlockSpec((1, D), lambda i: (0, 0)),
        ],
        out_specs=pl.BlockSpec((tb, D), lambda i: (i, 0)),
        out_shape=jax.ShapeDtypeStruct((n_tok, D), jnp.float32),
    )(rows, pos2d, mod_row, g2d, b2d)


def kernel(x, token_table, pos_emb, mod_table, gamma, beta):
    bsz, seq = x.shape
    n_tok = bsz * seq
    n_chunks = n_tok // NW // CHUNK
    x_arr = x.astype(jnp.int32).reshape(NW, n_chunks, CHUNK)
    rows = _make_sc_gather(n_tok)(x_arr, token_table)
    pos2d = pos_emb.reshape(seq, D)
    out = _tc_ln(rows, pos2d, mod_table[0:1], gamma.reshape(1, D),
                 beta.reshape(1, D), 1024, seq)
    return out.reshape(bsz, seq, D)


# speedup vs baseline: 4.8498x; 1.0164x over previous
import functools

import jax
import jax.numpy as jnp
from jax import lax
from jax.experimental import pallas as pl
from jax.experimental.pallas import tpu as pltpu
from jax.experimental.pallas import tpu_sc as plsc

D = 512
EPS = 1e-5
NW = 32
CHUNK = 64


def _make_sc_gather(n_tok):
    tok_per_w = n_tok // NW
    n_chunks = tok_per_w // CHUNK
    mesh = plsc.VectorSubcoreMesh(core_axis_name="c", subcore_axis_name="s")

    def body(x_hbm, table_hbm, out_hbm, idx_v, buf0, buf1, g0, g1, o0, o1):
        wid = lax.axis_index("s") * 2 + lax.axis_index("c")
        base = wid * tok_per_w
        pltpu.sync_copy(x_hbm.at[wid], idx_v)

        bufs = (buf0, buf1)
        gsems = (g0, g1)
        osems = (o0, o1)

        def gather(c):
            return pltpu.async_copy(
                table_hbm.at[idx_v.at[c]], bufs[c % 2], gsems[c % 2])

        def put(c):
            return pltpu.async_copy(
                bufs[c % 2], out_hbm.at[pl.ds(base + c * CHUNK, CHUNK)],
                osems[c % 2])

        gathers = [None] * n_chunks
        puts = [None] * n_chunks
        gathers[0] = gather(0)
        for c in range(n_chunks):
            if c + 1 < n_chunks:
                if c >= 1:
                    puts[c - 1].wait()
                gathers[c + 1] = gather(c + 1)
            gathers[c].wait()
            puts[c] = put(c)
        puts[n_chunks - 2].wait()
        puts[n_chunks - 1].wait()

    return pl.kernel(
        body,
        out_type=jax.ShapeDtypeStruct((n_tok, D), jnp.float32),
        mesh=mesh,
        scratch_types=[
            pltpu.VMEM((n_chunks, CHUNK), jnp.int32),
            pltpu.VMEM((CHUNK, D), jnp.float32),
            pltpu.VMEM((CHUNK, D), jnp.float32),
            pltpu.SemaphoreType.DMA,
            pltpu.SemaphoreType.DMA,
            pltpu.SemaphoreType.DMA,
            pltpu.SemaphoreType.DMA,
        ],
    )


def _tc_ln_body(rows_ref, pos_ref, mod_ref, g_ref, b_ref, o_ref):
    h = rows_ref[...] + pos_ref[...] + mod_ref[...]
    mean = jnp.mean(h, axis=-1, keepdims=True)
    c = h - mean
    var = jnp.mean(c * c, axis=-1, keepdims=True)
    o_ref[...] = c * lax.rsqrt(var + EPS) * g_ref[...] + b_ref[...]


def _tc_ln(rows, pos2d, mod_row, g2d, b2d, tb, seq):
    n_tok = rows.shape[0]
    pos_blocks = seq // tb
    return pl.pallas_call(
        _tc_ln_body,
        grid=(n_tok // tb,),
        in_specs=[
            pl.BlockSpec((tb, D), lambda i: (i, 0)),
            pl.BlockSpec((tb, D), lambda i: (i % pos_blocks, 0)),
            pl.BlockSpec((1, D), lambda i: (0, 0)),
            pl.BlockSpec((1, D), lambda i: (0, 0)),
            pl.BlockSpec((1, D), lambda i: (0, 0)),
        ],
        out_specs=pl.BlockSpec((tb, D), lambda i: (i, 0)),
        out_shape=jax.ShapeDtypeStruct((n_tok, D), jnp.float32),
    )(rows, pos2d, mod_row, g2d, b2d)


def kernel(x, token_table, pos_emb, mod_table, gamma, beta):
    bsz, seq = x.shape
    n_tok = bsz * seq
    n_chunks = n_tok // NW // CHUNK
    x_arr = x.astype(jnp.int32).reshape(NW, n_chunks, CHUNK)
    rows = _make_sc_gather(n_tok)(x_arr, token_table)
    pos2d = pos_emb.reshape(seq, D)
    out = _tc_ln(rows, pos2d, mod_table[0:1], gamma.reshape(1, D),
                 beta.reshape(1, D), 2048, seq)
    return out.reshape(bsz, seq, D)


# speedup vs baseline: 5.4140x; 1.1163x over previous
import functools

import jax
import jax.numpy as jnp
from jax import lax
from jax.experimental import pallas as pl
from jax.experimental.pallas import tpu as pltpu
from jax.experimental.pallas import tpu_sc as plsc

D = 512
EPS = 1e-5
NW = 32
CHUNK = 64


def _make_sc_gather(n_tok):
    tok_per_w = n_tok // NW
    n_chunks = tok_per_w // CHUNK
    mesh = plsc.VectorSubcoreMesh(core_axis_name="c", subcore_axis_name="s")

    def body(x_hbm, table_hbm, out_hbm, idx_v, buf0, buf1, g0, g1, o0, o1):
        wid = lax.axis_index("s") * 2 + lax.axis_index("c")
        base = wid * tok_per_w
        pltpu.sync_copy(x_hbm.at[wid], idx_v)

        bufs = (buf0, buf1)
        gsems = (g0, g1)
        osems = (o0, o1)

        def gather(c):
            return pltpu.async_copy(
                table_hbm.at[idx_v.at[c]], bufs[c % 2], gsems[c % 2])

        def put(c):
            return pltpu.async_copy(
                bufs[c % 2], out_hbm.at[pl.ds(base + c * CHUNK, CHUNK)],
                osems[c % 2])

        gathers = [None] * n_chunks
        puts = [None] * n_chunks
        gathers[0] = gather(0)
        for c in range(n_chunks):
            if c + 1 < n_chunks:
                if c >= 1:
                    puts[c - 1].wait()
                gathers[c + 1] = gather(c + 1)
            gathers[c].wait()
            puts[c] = put(c)
        puts[n_chunks - 2].wait()
        puts[n_chunks - 1].wait()

    return pl.kernel(
        body,
        out_type=jax.ShapeDtypeStruct((n_tok, D), jnp.float32),
        mesh=mesh,
        scratch_types=[
            pltpu.VMEM((n_chunks, CHUNK), jnp.int32),
            pltpu.VMEM((CHUNK, D), jnp.float32),
            pltpu.VMEM((CHUNK, D), jnp.float32),
            pltpu.SemaphoreType.DMA,
            pltpu.SemaphoreType.DMA,
            pltpu.SemaphoreType.DMA,
            pltpu.SemaphoreType.DMA,
        ],
    )


def _tc_ln_body(rows_ref, pos_ref, mod_ref, g_ref, b_ref, o_ref):
    h = rows_ref[...] + pos_ref[...][None] + mod_ref[...][None]
    mean = jnp.mean(h, axis=-1, keepdims=True)
    c = h - mean
    var = jnp.mean(c * c, axis=-1, keepdims=True)
    o_ref[...] = (c * lax.rsqrt(var + EPS) * g_ref[...][None]
                  + b_ref[...][None])


def _tc_ln(rows3d, pos2d, mod_row, g2d, b2d, ts, bsz, seq):
    return pl.pallas_call(
        _tc_ln_body,
        grid=(seq // ts,),
        in_specs=[
            pl.BlockSpec((bsz, ts, D), lambda j: (0, j, 0)),
            pl.BlockSpec((ts, D), lambda j: (j, 0)),
            pl.BlockSpec((1, D), lambda j: (0, 0)),
            pl.BlockSpec((1, D), lambda j: (0, 0)),
            pl.BlockSpec((1, D), lambda j: (0, 0)),
        ],
        out_specs=pl.BlockSpec((bsz, ts, D), lambda j: (0, j, 0)),
        out_shape=jax.ShapeDtypeStruct((bsz, seq, D), jnp.float32),
    )(rows3d, pos2d, mod_row, g2d, b2d)


def kernel(x, token_table, pos_emb, mod_table, gamma, beta):
    bsz, seq = x.shape
    n_tok = bsz * seq
    n_chunks = n_tok // NW // CHUNK
    x_arr = x.astype(jnp.int32).reshape(NW, n_chunks, CHUNK)
    rows = _make_sc_gather(n_tok)(x_arr, token_table)
    pos2d = pos_emb.reshape(seq, D)
    return _tc_ln(rows.reshape(bsz, seq, D), pos2d, mod_table[0:1],
                  gamma.reshape(1, D), beta.reshape(1, D), 1024, bsz, seq)
